# baseline (speedup 1.0000x reference)
import functools
import jax
import jax.numpy as jnp
from jax import lax
from jax.experimental import pallas as pl
from jax.experimental.pallas import tpu as pltpu
from jax.experimental.pallas import tpu_sc as plsc

N = 10000
E1 = 128000
E2 = 32000
NREL = 500
IN_DIM = 128
NHID = 64
D2 = 128
ALPHA = 0.2
R = 512
ZROW = 500
NP = 10240
NPR = 80
E1P = 131072
E2P = 32768
EOP = 163840
CH = 32
W = 128
NWORK = 32
HI = jax.lax.Precision.HIGHEST

f32 = jnp.float32
i32 = jnp.int32


def _dg(a, b):
    return lax.dot_general(a, b, (((1,), (1,)), ((), ())), precision=HI,
                           preferred_element_type=f32)


def _tc_dense(x, rel_ext, w1_0, w2_0, w1_1, w2_1, w1_out, w2_out, WR):
    def body(x_ref, rel_ref, w10_ref, w20_ref, w11_ref, w21_ref,
             w1o_ref, w2o_ref, wr_ref,
             ab_ref, ns_ref, relc_ref, relco_ref, outrel_ref, srel_ref):
        x = x_ref[...]
        rel = rel_ref[...]
        w10 = w10_ref[...]
        w11 = w11_ref[...]
        A0 = _dg(x, w10[:, :IN_DIM])
        A1 = _dg(x, w11[:, :IN_DIM])
        B0 = _dg(x, w10[:, IN_DIM:2 * IN_DIM])
        B1 = _dg(x, w11[:, IN_DIM:2 * IN_DIM])
        ab_ref[...] = jnp.concatenate([A0, A1, B0, B1], axis=1)
        ns_ref[...] = jnp.concatenate(
            [_dg(A0, w20_ref[...]), _dg(A1, w21_ref[...]),
             _dg(B0, w20_ref[...]), _dg(B1, w21_ref[...]),
             jnp.zeros((N, 124), f32)], axis=1)
        rc0 = _dg(rel, w10[:, 2 * IN_DIM:])
        rc1 = _dg(rel, w11[:, 2 * IN_DIM:])
        relc_ref[...] = jnp.concatenate([rc0, rc1], axis=1)
        outrel = lax.dot_general(rel, wr_ref[...], (((1,), (0,)), ((), ())),
                                 precision=HI, preferred_element_type=f32)
        outrel_ref[...] = outrel
        relco = _dg(outrel, w1o_ref[...][:, 2 * D2:])
        relco_ref[...] = relco
        srel_ref[0:1, :] = _dg(w20_ref[...], rc0)
        srel_ref[1:2, :] = _dg(w21_ref[...], rc1)
        srel_ref[2:3, :] = _dg(w2o_ref[...], relco)
        srel_ref[3:4, :] = jnp.zeros((1, R), f32)

    return pl.pallas_call(
        body,
        out_shape=[
            jax.ShapeDtypeStruct((N, 256), f32),
            jax.ShapeDtypeStruct((N, 128), f32),
            jax.ShapeDtypeStruct((R, 128), f32),
            jax.ShapeDtypeStruct((R, 128), f32),
            jax.ShapeDtypeStruct((R, 128), f32),
            jax.ShapeDtypeStruct((4, R), f32),
        ],
    )(x, rel_ext, w1_0, w2_0, w1_1, w2_1, w1_out, w2_out, WR)


def _tc_edgeproj(ee_pad, w1_0, w2_0, w1_1, w2_1):
    BE = 2048
    grid = E1P // BE

    def body(ee_ref, w10_ref, w20_ref, w11_ref, w21_ref, cl_ref, sc_ref):
        ee = ee_ref[...]
        c0 = _dg(ee, w10_ref[...][:, 2 * IN_DIM:])
        c1 = _dg(ee, w11_ref[...][:, 2 * IN_DIM:])
        cl_ref[...] = jnp.concatenate([c0, c1], axis=1)
        sc_ref[0:1, :] = _dg(w20_ref[...], c0)
        sc_ref[1:2, :] = _dg(w21_ref[...], c1)
        sc_ref[2:4, :] = jnp.zeros((2, BE), f32)

    return pl.pallas_call(
        body,
        grid=(grid,),
        in_specs=[
            pl.BlockSpec((BE, NHID), lambda i: (i, 0)),
            pl.BlockSpec((NHID, 2 * IN_DIM + NHID), lambda i: (0, 0)),
            pl.BlockSpec((1, NHID), lambda i: (0, 0)),
            pl.BlockSpec((NHID, 2 * IN_DIM + NHID), lambda i: (0, 0)),
            pl.BlockSpec((1, NHID), lambda i: (0, 0)),
        ],
        out_specs=[
            pl.BlockSpec((BE, 128), lambda i: (i, 0)),
            pl.BlockSpec((4, BE), lambda i: (0, i)),
        ],
        out_shape=[
            jax.ShapeDtypeStruct((E1P, 128), f32),
            jax.ShapeDtypeStruct((4, E1P), f32),
        ],
    )(ee_pad, w1_0, w2_0, w1_1, w2_1)


def _combine(Hacc, rsacc, Afused):
    H = Hacc[0] + Hacc[1]
    rs = jnp.sum(rsacc, axis=0)
    rs0 = rs[0, :N]
    rs1 = rs[1, :N]
    d0 = jnp.where(rs0 < 1e-12, 1e-12, rs0)[:, None]
    d1 = jnp.where(rs1 < 1e-12, 1e-12, rs1)[:, None]
    h0 = (Afused[:, :NHID] * rs0[:, None] + H[:N, :NHID]) / d0
    h1 = (Afused[:, NHID:] * rs1[:, None] + H[:N, NHID:128]) / d1
    return jnp.concatenate([h0, h1], axis=1)


def _tc_mid(Hacc, rsacc, Afused, w1_out, w2_out):
    def body(h_ref, rs_ref, a_ref, w1o_ref, w2o_ref, ao_ref, bo_ref, ns_ref):
        lx = _combine(h_ref[...], rs_ref[...], a_ref[...])
        w1o = w1o_ref[...]
        Ao = _dg(lx, w1o[:, :D2])
        Bo = _dg(lx, w1o[:, D2:2 * D2])
        ao_ref[...] = Ao
        bo_ref[...] = Bo
        sa = _dg(Ao, w2o_ref[...])
        sb = _dg(Bo, w2o_ref[...])
        ns_ref[...] = jnp.concatenate(
            [sa, sa, sb, sb, jnp.zeros((N, 124), f32)], axis=1)

    return pl.pallas_call(
        body,
        out_shape=[
            jax.ShapeDtypeStruct((N, 128), f32),
            jax.ShapeDtypeStruct((N, 128), f32),
            jax.ShapeDtypeStruct((N, 128), f32),
        ],
    )(Hacc, rsacc, Afused, w1_out, w2_out)


def _tc_final(Hacc, rsacc, Ao):
    def body(h_ref, rs_ref, a_ref, out_ref):
        H = h_ref[0] + h_ref[1]
        rs = jnp.sum(rs_ref[...], axis=0)[0, :N]
        d = jnp.where(rs < 1e-12, 1e-12, rs)[:, None]
        h = (a_ref[...] * rs[:, None] + H[:N, :128]) / d
        h = jnp.where(h > 0, h, jnp.exp(h) - 1.0)
        out_ref[...] = jnp.where(h > 0, h, jnp.exp(h) - 1.0)

    return pl.pallas_call(
        body, out_shape=jax.ShapeDtypeStruct((N, 128), f32),
    )(Hacc, rsacc, Ao)


def _make_sc_pass(n_l, n_g):
    ncl = n_l // CH
    ncg = n_g // CH
    mesh = plsc.VectorSubcoreMesh(core_axis_name="c", subcore_axis_name="s")
    rows_per_tile = NP // 16

    def kern(srcl_h, dstl_h, scl_h, cl_h, srcg_h, dstg_h, g0_h, g1_h,
             relc_h, srel_h, b_h, ns_h, zeros_h, idx01_h, zrs_h,
             out_h, rs_h,
             spmem, rs_spmem, srcv, dstv, g0v, g1v, sc0v, sc1v, eeb0, eeb1,
             bg, cg1, cg2, outv, nsv, ndv, srv, rsv, idx01v, sem):
        c = lax.axis_index("c")
        s = lax.axis_index("s")
        w = c * 16 + s

        pltpu.sync_copy(zeros_h.at[pl.ds(s * rows_per_tile, rows_per_tile)],
                        spmem.at[pl.ds(s * rows_per_tile, rows_per_tile)])
        @pl.when(s == 0)
        def _():
            pltpu.sync_copy(zrs_h, rs_spmem)
        pltpu.sync_copy(srel_h, srv)
        pltpu.sync_copy(idx01_h, idx01v)
        plsc.subcore_barrier()

        zero16 = jnp.zeros((16,), i32)
        one16 = jnp.full((16,), 1, i32)
        two16 = jnp.full((16,), 2, i32)
        three16 = jnp.full((16,), 3, i32)
        lane = lax.iota(i32, 16)
        pltpu.sync_copy(zrs_h, rsv)

        def attention(g, sc0, sc1):
            srcg = plsc.load_gather(srcv, [g * 16 + lane])
            sh = lax.shift_right_logical(srcg, 7)
            sl_ = jnp.bitwise_and(srcg, 127)
            row = g * 16 + lane
            sa0 = plsc.load_gather(nsv, [row, zero16])
            sa1 = plsc.load_gather(nsv, [row, one16])
            sb0 = plsc.load_gather(ndv, [row, two16])
            sb1 = plsc.load_gather(ndv, [row, three16])
            z0 = sa0 + sb0 + sc0
            z1 = sa1 + sb1 + sc1
            ee0 = jnp.exp(jnp.where(z0 >= 0, -z0, -ALPHA * z0))
            ee1 = jnp.exp(jnp.where(z1 >= 0, -z1, -ALPHA * z1))
            eeb0[...] = ee0
            eeb1[...] = ee1
            plsc.addupdate_scatter(rsv, [zero16, sh, sl_], ee0)
            plsc.addupdate_scatter(rsv, [one16, sh, sl_], ee1)

        def rows_body(g, two_c):
            def lbody(l, _):
                row = g * 16 + l
                lidx = jnp.full((16,), l, i32)
                e0 = plsc.load_gather(eeb0, [lidx])
                e1 = plsc.load_gather(eeb1, [lidx])
                for jj in range(8):
                    sl = pl.ds(jj * 16, 16)
                    crow = cg1[row, sl]
                    if two_c:
                        crow = crow + cg2[row, sl]
                    ee = e0 if jj < 4 else e1
                    outv[row, sl] = ee * (bg[row, sl] + crow)
                return 0
            lax.fori_loop(0, 16, lbody, 0, unroll=True)

        def chunk_l(i, _):
            base = w * n_l + i * CH
            pltpu.sync_copy(srcl_h.at[pl.ds(base, CH)], srcv)
            pltpu.sync_copy(dstl_h.at[pl.ds(base, CH)], dstv)
            pltpu.sync_copy(scl_h.at[0, pl.ds(base, CH)], sc0v)
            pltpu.sync_copy(scl_h.at[1, pl.ds(base, CH)], sc1v)
            pltpu.async_copy(b_h.at[dstv], bg, sem).wait()
            pltpu.async_copy(ns_h.at[srcv], nsv, sem).wait()
            pltpu.async_copy(ns_h.at[dstv], ndv, sem).wait()
            pltpu.sync_copy(cl_h.at[pl.ds(base, CH)], cg1)

            def gbody(g, _):
                attention(g,
                          plsc.load_gather(sc0v, [g * 16 + lane]),
                          plsc.load_gather(sc1v, [g * 16 + lane]))
                rows_body(g, False)
                return 0
            lax.fori_loop(0, CH // 16, gbody, 0)
            pltpu.sync_copy(outv, spmem.at[srcv], add=True)
            return 0

        def chunk_g(i, _):
            base = w * n_g + i * CH
            pltpu.sync_copy(srcg_h.at[pl.ds(base, CH)], srcv)
            pltpu.sync_copy(dstg_h.at[pl.ds(base, CH)], dstv)
            pltpu.sync_copy(g0_h.at[pl.ds(base, CH)], g0v)
            pltpu.sync_copy(g1_h.at[pl.ds(base, CH)], g1v)
            pltpu.async_copy(b_h.at[dstv], bg, sem).wait()
            pltpu.async_copy(ns_h.at[srcv], nsv, sem).wait()
            pltpu.async_copy(ns_h.at[dstv], ndv, sem).wait()
            pltpu.async_copy(relc_h.at[g0v], cg1, sem).wait()
            pltpu.async_copy(relc_h.at[g1v], cg2, sem).wait()

            def gbody(g, _):
                t0 = plsc.load_gather(g0v, [g * 16 + lane])
                t1 = plsc.load_gather(g1v, [g * 16 + lane])
                t0h = lax.shift_right_logical(t0, 7)
                t0l = jnp.bitwise_and(t0, 127)
                t1h = lax.shift_right_logical(t1, 7)
                t1l = jnp.bitwise_and(t1, 127)
                sc0 = (plsc.load_gather(srv, [zero16, t0h, t0l])
                       + plsc.load_gather(srv, [zero16, t1h, t1l]))
                sc1 = (plsc.load_gather(srv, [one16, t0h, t0l])
                       + plsc.load_gather(srv, [one16, t1h, t1l]))
                attention(g, sc0, sc1)
                rows_body(g, True)
                return 0
            lax.fori_loop(0, CH // 16, gbody, 0)
            pltpu.sync_copy(outv, spmem.at[srcv], add=True)
            return 0

        if ncl > 0:
            lax.fori_loop(0, ncl, chunk_l, 0)
        lax.fori_loop(0, ncg, chunk_g, 0)
        pltpu.sync_copy(rsv, rs_spmem.at[idx01v], add=True)
        plsc.subcore_barrier()
        pltpu.sync_copy(spmem.at[pl.ds(s * rows_per_tile, rows_per_tile)],
                        out_h.at[c, pl.ds(s * rows_per_tile, rows_per_tile)])
        @pl.when(s == 0)
        def _():
            pltpu.sync_copy(rs_spmem, rs_h.at[c])

    return pl.kernel(
        kern, mesh=mesh,
        compiler_params=pltpu.CompilerParams(needs_layout_passes=False),
        out_type=[jax.ShapeDtypeStruct((2, NP, W), f32),
                  jax.ShapeDtypeStruct((2, 2, NPR, 128), f32)],
        scratch_types=[
            pltpu.VMEM_SHARED((NP, W), f32),
            pltpu.VMEM_SHARED((2, NPR, 128), f32),
            pltpu.VMEM((CH,), i32),
            pltpu.VMEM((CH,), i32),
            pltpu.VMEM((CH,), i32),
            pltpu.VMEM((CH,), i32),
            pltpu.VMEM((CH,), f32),
            pltpu.VMEM((CH,), f32),
            pltpu.VMEM((16,), f32),
            pltpu.VMEM((16,), f32),
            pltpu.VMEM((CH, 128), f32),
            pltpu.VMEM((CH, 128), f32),
            pltpu.VMEM((CH, 128), f32),
            pltpu.VMEM((CH, W), f32),
            pltpu.VMEM((CH, 128), f32),
            pltpu.VMEM((CH, 128), f32),
            pltpu.VMEM((2, R // 128, 128), f32),
            pltpu.VMEM((2, NPR, 128), f32),
            pltpu.VMEM((2,), i32),
            pltpu.SemaphoreType.DMA,
        ],
    )


def kernel(term_embeddings, relation_embeddings, edge_list, edge_type,
           edge_embed, edge_list_nhop, edge_type_nhop, CUDA,
           w1_0, w2_0, w1_1, w2_1, w1_out, w2_out, WR):
    x = term_embeddings
    rel_ext = jnp.pad(relation_embeddings, ((0, R - NREL), (0, 0)))

    AB, nodescal, relC, relCo, outrel_ext, srel4 = _tc_dense(
        x, rel_ext, w1_0, w2_0, w1_1, w2_1, w1_out, w2_out, WR)
    Afused = AB[:, :128]
    Bfused = AB[:, 128:]

    ee_pad = jnp.pad(edge_embed, ((0, E1P - E1), (0, 0)))
    CL, sCL4 = _tc_edgeproj(ee_pad, w1_0, w2_0, w1_1, w2_1)

    padi = lambda a, L, v: jnp.pad(a, (0, L - a.shape[0]), constant_values=v)

    srcL = padi(edge_list[0], E1P, N)
    dstL = padi(edge_list[1], E1P, 0)
    t0 = edge_type_nhop[:, 0]
    t1 = edge_type_nhop[:, 1]
    srcGh = padi(edge_list_nhop[0], E2P, N)
    dstGh = padi(edge_list_nhop[1], E2P, 0)
    g0h = padi(t0, E2P, ZROW)
    g1h = padi(t1, E2P, ZROW)
    zeros_buf = jnp.zeros((NP, W), f32)
    idx01 = jnp.arange(2, dtype=i32)
    zrs = jnp.zeros((2, NPR, 128), f32)
    ns_pad = jnp.pad(nodescal, ((0, NP - N), (0, 0)))

    sc_heads = _make_sc_pass(E1P // NWORK, E2P // NWORK)
    Hacc, rsacc = sc_heads(srcL, dstL, sCL4[0:2], CL, srcGh, dstGh, g0h, g1h,
                           relC, srel4[0:2].reshape(2, R // 128, 128), Bfused,
                           ns_pad, zeros_buf, idx01, zrs)

    Ao, Bo, nso = _tc_mid(Hacc, rsacc.reshape(2, 2, NP), Afused, w1_out, w2_out)

    srcO = jnp.concatenate([edge_list[0], edge_list_nhop[0],
                            jnp.full((EOP - E1 - E2,), N, i32)])
    dstO = jnp.concatenate([edge_list[1], edge_list_nhop[1],
                            jnp.zeros((EOP - E1 - E2,), i32)])
    g0o = jnp.concatenate([edge_type, t0,
                           jnp.full((EOP - E1 - E2,), ZROW, i32)])
    g1o = jnp.concatenate([jnp.full((E1,), ZROW, i32), t1,
                           jnp.full((EOP - E1 - E2,), ZROW, i32)])
    srel_o = jnp.concatenate([srel4[2:3], srel4[2:3]],
                             axis=0).reshape(2, R // 128, 128)
    nso_pad = jnp.pad(nso, ((0, NP - N), (0, 0)))

    dummy_i = jnp.zeros((8,), i32)
    dummy_s = jnp.zeros((2, 8), f32)
    dummy_c = jnp.zeros((8, 128), f32)
    sc_out = _make_sc_pass(0, EOP // NWORK)
    Hout, rso = sc_out(dummy_i, dummy_i, dummy_s, dummy_c, srcO, dstO, g0o, g1o,
                       relCo, srel_o, Bo, nso_pad, zeros_buf, idx01, zrs)

    out_entity = _tc_final(Hout, rso.reshape(2, 2, NP), Ao)
    out_relation = outrel_ext[:NREL]
    return (out_entity, out_relation)


# speedup vs baseline: 1.1228x; 1.1228x over previous
import functools
import jax
import jax.numpy as jnp
from jax import lax
from jax.experimental import pallas as pl
from jax.experimental.pallas import tpu as pltpu
from jax.experimental.pallas import tpu_sc as plsc

N = 10000
E1 = 128000
E2 = 32000
NREL = 500
IN_DIM = 128
NHID = 64
D2 = 128
ALPHA = 0.2
R = 512
ZROW = 500
NP = 10240
NPR = 80
E1P = 131072
E2P = 32768
EOP = 163840
CH = 32
W = 128
NWORK = 32
HI = jax.lax.Precision.HIGHEST

f32 = jnp.float32
i32 = jnp.int32


def _dg(a, b):
    return lax.dot_general(a, b, (((1,), (1,)), ((), ())), precision=HI,
                           preferred_element_type=f32)


def _tc_dense(x, rel_ext, w1_0, w2_0, w1_1, w2_1, w1_out, w2_out, WR):
    def body(x_ref, rel_ref, w10_ref, w20_ref, w11_ref, w21_ref,
             w1o_ref, w2o_ref, wr_ref,
             ab_ref, ns_ref, relc_ref, relco_ref, outrel_ref, srel_ref):
        x = x_ref[...]
        rel = rel_ref[...]
        w10 = w10_ref[...]
        w11 = w11_ref[...]
        A0 = _dg(x, w10[:, :IN_DIM])
        A1 = _dg(x, w11[:, :IN_DIM])
        B0 = _dg(x, w10[:, IN_DIM:2 * IN_DIM])
        B1 = _dg(x, w11[:, IN_DIM:2 * IN_DIM])
        ab_ref[...] = jnp.concatenate([A0, A1, B0, B1], axis=1)
        ns_ref[...] = jnp.concatenate(
            [_dg(A0, w20_ref[...]), _dg(A1, w21_ref[...]),
             _dg(B0, w20_ref[...]), _dg(B1, w21_ref[...]),
             jnp.zeros((N, 124), f32)], axis=1)
        rc0 = _dg(rel, w10[:, 2 * IN_DIM:])
        rc1 = _dg(rel, w11[:, 2 * IN_DIM:])
        relc_ref[...] = jnp.concatenate([rc0, rc1], axis=1)
        outrel = lax.dot_general(rel, wr_ref[...], (((1,), (0,)), ((), ())),
                                 precision=HI, preferred_element_type=f32)
        outrel_ref[...] = outrel
        relco = _dg(outrel, w1o_ref[...][:, 2 * D2:])
        relco_ref[...] = relco
        srel_ref[0:1, :] = _dg(w20_ref[...], rc0)
        srel_ref[1:2, :] = _dg(w21_ref[...], rc1)
        srel_ref[2:3, :] = _dg(w2o_ref[...], relco)
        srel_ref[3:4, :] = jnp.zeros((1, R), f32)

    return pl.pallas_call(
        body,
        out_shape=[
            jax.ShapeDtypeStruct((N, 256), f32),
            jax.ShapeDtypeStruct((N, 128), f32),
            jax.ShapeDtypeStruct((R, 128), f32),
            jax.ShapeDtypeStruct((R, 128), f32),
            jax.ShapeDtypeStruct((R, 128), f32),
            jax.ShapeDtypeStruct((4, R), f32),
        ],
    )(x, rel_ext, w1_0, w2_0, w1_1, w2_1, w1_out, w2_out, WR)


def _tc_edgeproj(ee_pad, w1_0, w2_0, w1_1, w2_1):
    BE = 2048
    grid = E1P // BE

    def body(ee_ref, w10_ref, w20_ref, w11_ref, w21_ref, cl_ref, sc_ref):
        ee = ee_ref[...]
        c0 = _dg(ee, w10_ref[...][:, 2 * IN_DIM:])
        c1 = _dg(ee, w11_ref[...][:, 2 * IN_DIM:])
        cl_ref[...] = jnp.concatenate([c0, c1], axis=1)
        sc_ref[0:1, :] = _dg(w20_ref[...], c0)
        sc_ref[1:2, :] = _dg(w21_ref[...], c1)
        sc_ref[2:4, :] = jnp.zeros((2, BE), f32)

    return pl.pallas_call(
        body,
        grid=(grid,),
        in_specs=[
            pl.BlockSpec((BE, NHID), lambda i: (i, 0)),
            pl.BlockSpec((NHID, 2 * IN_DIM + NHID), lambda i: (0, 0)),
            pl.BlockSpec((1, NHID), lambda i: (0, 0)),
            pl.BlockSpec((NHID, 2 * IN_DIM + NHID), lambda i: (0, 0)),
            pl.BlockSpec((1, NHID), lambda i: (0, 0)),
        ],
        out_specs=[
            pl.BlockSpec((BE, 128), lambda i: (i, 0)),
            pl.BlockSpec((4, BE), lambda i: (0, i)),
        ],
        out_shape=[
            jax.ShapeDtypeStruct((E1P, 128), f32),
            jax.ShapeDtypeStruct((4, E1P), f32),
        ],
    )(ee_pad, w1_0, w2_0, w1_1, w2_1)


def _combine(Hacc, rsacc, Afused):
    H = Hacc[0] + Hacc[1]
    rs = jnp.sum(rsacc, axis=0)
    rs0 = rs[0, :N]
    rs1 = rs[1, :N]
    d0 = jnp.where(rs0 < 1e-12, 1e-12, rs0)[:, None]
    d1 = jnp.where(rs1 < 1e-12, 1e-12, rs1)[:, None]
    h0 = (Afused[:, :NHID] * rs0[:, None] + H[:N, :NHID]) / d0
    h1 = (Afused[:, NHID:] * rs1[:, None] + H[:N, NHID:128]) / d1
    return jnp.concatenate([h0, h1], axis=1)


def _tc_mid(Hacc, rsacc, Afused, w1_out, w2_out):
    def body(h_ref, rs_ref, a_ref, w1o_ref, w2o_ref, ao_ref, bo_ref, ns_ref):
        lx = _combine(h_ref[...], rs_ref[...], a_ref[...])
        w1o = w1o_ref[...]
        Ao = _dg(lx, w1o[:, :D2])
        Bo = _dg(lx, w1o[:, D2:2 * D2])
        ao_ref[...] = Ao
        bo_ref[...] = Bo
        sa = _dg(Ao, w2o_ref[...])
        sb = _dg(Bo, w2o_ref[...])
        ns_ref[...] = jnp.concatenate(
            [sa, sa, sb, sb, jnp.zeros((N, 124), f32)], axis=1)

    return pl.pallas_call(
        body,
        out_shape=[
            jax.ShapeDtypeStruct((N, 128), f32),
            jax.ShapeDtypeStruct((N, 128), f32),
            jax.ShapeDtypeStruct((N, 128), f32),
        ],
    )(Hacc, rsacc, Afused, w1_out, w2_out)


def _tc_final(Hacc, rsacc, Ao):
    def body(h_ref, rs_ref, a_ref, out_ref):
        H = h_ref[0] + h_ref[1]
        rs = jnp.sum(rs_ref[...], axis=0)[0, :N]
        d = jnp.where(rs < 1e-12, 1e-12, rs)[:, None]
        h = (a_ref[...] * rs[:, None] + H[:N, :128]) / d
        h = jnp.where(h > 0, h, jnp.exp(h) - 1.0)
        out_ref[...] = jnp.where(h > 0, h, jnp.exp(h) - 1.0)

    return pl.pallas_call(
        body, out_shape=jax.ShapeDtypeStruct((N, 128), f32),
    )(Hacc, rsacc, Ao)


def _make_sc_pass(n_l, n_g):
    ncl = n_l // CH
    ncg = n_g // CH
    mesh = plsc.VectorSubcoreMesh(core_axis_name="c", subcore_axis_name="s")
    rows_per_tile = NP // 16

    def kern(srcl_h, dstl_h, scl_h, cl_h, srcg_h, dstg_h, g0_h, g1_h,
             relc_h, srel_h, b_h, ns_h, zeros_h, idx01_h, zrs_h,
             out_h, rs_h,
             spmem, rs_spmem, srcv, dstv, g0v, g1v, sc0v, sc1v, eeb0, eeb1,
             bg, cg1, cg2, outv, nsv, ndv, srv, rsv, idx01v, sem):
        c = lax.axis_index("c")
        s = lax.axis_index("s")
        w = c * 16 + s

        pltpu.sync_copy(zeros_h.at[pl.ds(s * rows_per_tile, rows_per_tile)],
                        spmem.at[pl.ds(s * rows_per_tile, rows_per_tile)])
        @pl.when(s == 0)
        def _():
            pltpu.sync_copy(zrs_h, rs_spmem)
        pltpu.sync_copy(srel_h, srv)
        pltpu.sync_copy(idx01_h, idx01v)
        plsc.subcore_barrier()

        zero16 = jnp.zeros((16,), i32)
        one16 = jnp.full((16,), 1, i32)
        two16 = jnp.full((16,), 2, i32)
        three16 = jnp.full((16,), 3, i32)
        lane = lax.iota(i32, 16)
        pltpu.sync_copy(zrs_h, rsv)

        def attention(g, sc0, sc1):
            srcg = plsc.load_gather(srcv, [g * 16 + lane])
            sh = lax.shift_right_logical(srcg, 7)
            sl_ = jnp.bitwise_and(srcg, 127)
            row = g * 16 + lane
            sa0 = plsc.load_gather(nsv, [row, zero16])
            sa1 = plsc.load_gather(nsv, [row, one16])
            sb0 = plsc.load_gather(ndv, [row, two16])
            sb1 = plsc.load_gather(ndv, [row, three16])
            z0 = sa0 + sb0 + sc0
            z1 = sa1 + sb1 + sc1
            ee0 = jnp.exp(jnp.where(z0 >= 0, -z0, -ALPHA * z0))
            ee1 = jnp.exp(jnp.where(z1 >= 0, -z1, -ALPHA * z1))
            eeb0[...] = ee0
            eeb1[...] = ee1
            plsc.addupdate_scatter(rsv, [zero16, sh, sl_], ee0)
            plsc.addupdate_scatter(rsv, [one16, sh, sl_], ee1)

        def rows_body(g, two_c):
            def lbody(l, _):
                row = g * 16 + l
                lidx = jnp.full((16,), l, i32)
                e0 = plsc.load_gather(eeb0, [lidx])
                e1 = plsc.load_gather(eeb1, [lidx])
                for jj in range(8):
                    sl = pl.ds(jj * 16, 16)
                    crow = cg1[row, sl]
                    if two_c:
                        crow = crow + cg2[row, sl]
                    ee = e0 if jj < 4 else e1
                    outv[row, sl] = ee * (bg[row, sl] + crow)
                return 0
            lax.fori_loop(0, 16, lbody, 0, unroll=True)

        def chunk_l(i, _):
            base = w * n_l + i * CH
            c1 = pltpu.async_copy(srcl_h.at[pl.ds(base, CH)], srcv, sem)
            c2 = pltpu.async_copy(dstl_h.at[pl.ds(base, CH)], dstv, sem)
            c3 = pltpu.async_copy(scl_h.at[0, pl.ds(base, CH)], sc0v, sem)
            c4 = pltpu.async_copy(scl_h.at[1, pl.ds(base, CH)], sc1v, sem)
            c5 = pltpu.async_copy(cl_h.at[pl.ds(base, CH)], cg1, sem)
            c1.wait()
            c2.wait()
            g1_ = pltpu.async_copy(b_h.at[dstv], bg, sem)
            g2_ = pltpu.async_copy(ns_h.at[srcv], nsv, sem)
            g3_ = pltpu.async_copy(ns_h.at[dstv], ndv, sem)
            c3.wait()
            c4.wait()
            c5.wait()
            g1_.wait()
            g2_.wait()
            g3_.wait()

            def gbody(g, _):
                attention(g,
                          plsc.load_gather(sc0v, [g * 16 + lane]),
                          plsc.load_gather(sc1v, [g * 16 + lane]))
                rows_body(g, False)
                return 0
            lax.fori_loop(0, CH // 16, gbody, 0)
            pltpu.sync_copy(outv, spmem.at[srcv], add=True)
            return 0

        def chunk_g(i, _):
            base = w * n_g + i * CH
            c1 = pltpu.async_copy(srcg_h.at[pl.ds(base, CH)], srcv, sem)
            c2 = pltpu.async_copy(dstg_h.at[pl.ds(base, CH)], dstv, sem)
            c3 = pltpu.async_copy(g0_h.at[pl.ds(base, CH)], g0v, sem)
            c4 = pltpu.async_copy(g1_h.at[pl.ds(base, CH)], g1v, sem)
            c1.wait()
            c2.wait()
            c3.wait()
            c4.wait()
            g1_ = pltpu.async_copy(b_h.at[dstv], bg, sem)
            g2_ = pltpu.async_copy(ns_h.at[srcv], nsv, sem)
            g3_ = pltpu.async_copy(ns_h.at[dstv], ndv, sem)
            g4_ = pltpu.async_copy(relc_h.at[g0v], cg1, sem)
            g5_ = pltpu.async_copy(relc_h.at[g1v], cg2, sem)
            g1_.wait()
            g2_.wait()
            g3_.wait()
            g4_.wait()
            g5_.wait()

            def gbody(g, _):
                t0 = plsc.load_gather(g0v, [g * 16 + lane])
                t1 = plsc.load_gather(g1v, [g * 16 + lane])
                t0h = lax.shift_right_logical(t0, 7)
                t0l = jnp.bitwise_and(t0, 127)
                t1h = lax.shift_right_logical(t1, 7)
                t1l = jnp.bitwise_and(t1, 127)
                sc0 = (plsc.load_gather(srv, [zero16, t0h, t0l])
                       + plsc.load_gather(srv, [zero16, t1h, t1l]))
                sc1 = (plsc.load_gather(srv, [one16, t0h, t0l])
                       + plsc.load_gather(srv, [one16, t1h, t1l]))
                attention(g, sc0, sc1)
                rows_body(g, True)
                return 0
            lax.fori_loop(0, CH // 16, gbody, 0)
            pltpu.sync_copy(outv, spmem.at[srcv], add=True)
            return 0

        if ncl > 0:
            lax.fori_loop(0, ncl, chunk_l, 0)
        lax.fori_loop(0, ncg, chunk_g, 0)
        pltpu.sync_copy(rsv, rs_spmem.at[idx01v], add=True)
        plsc.subcore_barrier()
        pltpu.sync_copy(spmem.at[pl.ds(s * rows_per_tile, rows_per_tile)],
                        out_h.at[c, pl.ds(s * rows_per_tile, rows_per_tile)])
        @pl.when(s == 0)
        def _():
            pltpu.sync_copy(rs_spmem, rs_h.at[c])

    return pl.kernel(
        kern, mesh=mesh,
        compiler_params=pltpu.CompilerParams(needs_layout_passes=False),
        out_type=[jax.ShapeDtypeStruct((2, NP, W), f32),
                  jax.ShapeDtypeStruct((2, 2, NPR, 128), f32)],
        scratch_types=[
            pltpu.VMEM_SHARED((NP, W), f32),
            pltpu.VMEM_SHARED((2, NPR, 128), f32),
            pltpu.VMEM((CH,), i32),
            pltpu.VMEM((CH,), i32),
            pltpu.VMEM((CH,), i32),
            pltpu.VMEM((CH,), i32),
            pltpu.VMEM((CH,), f32),
            pltpu.VMEM((CH,), f32),
            pltpu.VMEM((16,), f32),
            pltpu.VMEM((16,), f32),
            pltpu.VMEM((CH, 128), f32),
            pltpu.VMEM((CH, 128), f32),
            pltpu.VMEM((CH, 128), f32),
            pltpu.VMEM((CH, W), f32),
            pltpu.VMEM((CH, 128), f32),
            pltpu.VMEM((CH, 128), f32),
            pltpu.VMEM((2, R // 128, 128), f32),
            pltpu.VMEM((2, NPR, 128), f32),
            pltpu.VMEM((2,), i32),
            pltpu.SemaphoreType.DMA,
        ],
    )


def kernel(term_embeddings, relation_embeddings, edge_list, edge_type,
           edge_embed, edge_list_nhop, edge_type_nhop, CUDA,
           w1_0, w2_0, w1_1, w2_1, w1_out, w2_out, WR):
    x = term_embeddings
    rel_ext = jnp.pad(relation_embeddings, ((0, R - NREL), (0, 0)))

    AB, nodescal, relC, relCo, outrel_ext, srel4 = _tc_dense(
        x, rel_ext, w1_0, w2_0, w1_1, w2_1, w1_out, w2_out, WR)
    Afused = AB[:, :128]
    Bfused = AB[:, 128:]

    ee_pad = jnp.pad(edge_embed, ((0, E1P - E1), (0, 0)))
    CL, sCL4 = _tc_edgeproj(ee_pad, w1_0, w2_0, w1_1, w2_1)

    padi = lambda a, L, v: jnp.pad(a, (0, L - a.shape[0]), constant_values=v)

    srcL = padi(edge_list[0], E1P, N)
    dstL = padi(edge_list[1], E1P, 0)
    t0 = edge_type_nhop[:, 0]
    t1 = edge_type_nhop[:, 1]
    srcGh = padi(edge_list_nhop[0], E2P, N)
    dstGh = padi(edge_list_nhop[1], E2P, 0)
    g0h = padi(t0, E2P, ZROW)
    g1h = padi(t1, E2P, ZROW)
    zeros_buf = jnp.zeros((NP, W), f32)
    idx01 = jnp.arange(2, dtype=i32)
    zrs = jnp.zeros((2, NPR, 128), f32)
    ns_pad = jnp.pad(nodescal, ((0, NP - N), (0, 0)))

    sc_heads = _make_sc_pass(E1P // NWORK, E2P // NWORK)
    Hacc, rsacc = sc_heads(srcL, dstL, sCL4[0:2], CL, srcGh, dstGh, g0h, g1h,
                           relC, srel4[0:2].reshape(2, R // 128, 128), Bfused,
                           ns_pad, zeros_buf, idx01, zrs)

    Ao, Bo, nso = _tc_mid(Hacc, rsacc.reshape(2, 2, NP), Afused, w1_out, w2_out)

    srcO = jnp.concatenate([edge_list[0], edge_list_nhop[0],
                            jnp.full((EOP - E1 - E2,), N, i32)])
    dstO = jnp.concatenate([edge_list[1], edge_list_nhop[1],
                            jnp.zeros((EOP - E1 - E2,), i32)])
    g0o = jnp.concatenate([edge_type, t0,
                           jnp.full((EOP - E1 - E2,), ZROW, i32)])
    g1o = jnp.concatenate([jnp.full((E1,), ZROW, i32), t1,
                           jnp.full((EOP - E1 - E2,), ZROW, i32)])
    srel_o = jnp.concatenate([srel4[2:3], srel4[2:3]],
                             axis=0).reshape(2, R // 128, 128)
    nso_pad = jnp.pad(nso, ((0, NP - N), (0, 0)))

    dummy_i = jnp.zeros((8,), i32)
    dummy_s = jnp.zeros((2, 8), f32)
    dummy_c = jnp.zeros((8, 128), f32)
    sc_out = _make_sc_pass(0, EOP // NWORK)
    Hout, rso = sc_out(dummy_i, dummy_i, dummy_s, dummy_c, srcO, dstO, g0o, g1o,
                       relCo, srel_o, Bo, nso_pad, zeros_buf, idx01, zrs)

    out_entity = _tc_final(Hout, rso.reshape(2, 2, NP), Ao)
    out_relation = outrel_ext[:NREL]
    return (out_entity, out_relation)
